# Initial kernel scaffold; baseline (speedup 1.0000x reference)
import jax
import jax.numpy as jnp
from jax.experimental import pallas as pl


def kernel(h_V, h_E, edge_idx, W1_w, W1_b, W2_w, W2_b, W3_w, W3_b, d1_w, d1_b, d2_w, d2_b, ln1_g, ln1_b, ln2_g, ln2_b):
    raise NotImplementedError("write your pallas kernel here")


# speedup vs baseline: 2.4372x; 2.4372x over previous
import functools

import jax
import jax.numpy as jnp
from jax import lax
from jax.experimental import pallas as pl
from jax.experimental.pallas import tpu as pltpu
from jax.experimental.pallas import tpu_sc as plsc

N_NODES = 10000
N_EDGES = 320000
H = 128
NIN = 16
SCALE_INV = 1.0 / 30.0

NUM_CORES = 2
NUM_SUBCORES = 16
NW = NUM_CORES * NUM_SUBCORES
EDGES_PER_TILE = N_EDGES // NW
CHUNK = 80
NCHUNKS = EDGES_PER_TILE // CHUNK
NODE_A = 624
NODE_LAST = N_NODES - (NUM_SUBCORES - 1) * NODE_A

_SQRT_HALF = 0.7071067811865476


def _erf(x):
    a1 = 0.254829592
    a2 = -0.284496736
    a3 = 1.421413741
    a4 = -1.453152027
    a5 = 1.061405429
    p = 0.3275911
    ax = jnp.abs(x)
    t = 1.0 / (1.0 + p * ax)
    poly = ((((a5 * t + a4) * t + a3) * t + a2) * t + a1) * t
    y = 1.0 - poly * jnp.exp(-ax * ax)
    return jnp.where(x < 0, -y, y)


def _gelu(x):
    return 0.5 * x * (1.0 + _erf(x * _SQRT_HALF))


def _ln(x, g, b, eps=1e-5):
    mu = jnp.mean(x, axis=-1, keepdims=True)
    var = jnp.mean((x - mu) ** 2, axis=-1, keepdims=True)
    return (x - mu) * lax.rsqrt(var + eps) * g + b


EBLK = 3200


def _edge_body(x_ref, w1_ref, b1_ref, w2_ref, b2_ref, w3_ref, b3_ref, o_ref):
    x = x_ref[...]
    m = _gelu(jnp.dot(x, w1_ref[...], preferred_element_type=jnp.float32)
              + b1_ref[...])
    m = _gelu(jnp.dot(m, w2_ref[...], preferred_element_type=jnp.float32)
              + b2_ref[...])
    o_ref[...] = (jnp.dot(m, w3_ref[...], preferred_element_type=jnp.float32)
                  + b3_ref[...])


def _edge_mlp(h_E, W1_w, W1_b, W2_w, W2_b, W3_w, W3_b):
    full = lambda shape: pl.BlockSpec(shape, lambda i: (0, 0))
    return pl.pallas_call(
        _edge_body,
        grid=(N_EDGES // EBLK,),
        in_specs=[
            pl.BlockSpec((EBLK, H + NIN), lambda i: (i, 0)),
            full((H + NIN, H)), full((1, H)),
            full((H, H)), full((1, H)),
            full((H, H)), full((1, H)),
        ],
        out_specs=pl.BlockSpec((EBLK, H), lambda i: (i, 0)),
        out_shape=jax.ShapeDtypeStruct((N_EDGES, H), jnp.float32),
    )(h_E, W1_w, W1_b.reshape(1, H), W2_w, W2_b.reshape(1, H),
      W3_w, W3_b.reshape(1, H))


@functools.lru_cache(maxsize=1)
def _make_scatter_sum():
    mesh = plsc.VectorSubcoreMesh(core_axis_name="c", subcore_axis_name="s")

    @functools.partial(
        pl.kernel,
        mesh=mesh,
        out_type=jax.ShapeDtypeStruct((NUM_CORES, N_NODES, H), jnp.float32),
        scratch_types=[
            pltpu.VMEM((NCHUNKS, CHUNK), jnp.int32),
            pltpu.VMEM((CHUNK, H), jnp.float32),
            pltpu.VMEM_SHARED((N_NODES, H), jnp.float32),
        ],
    )
    def _scatter_sum(msg_hbm, src_hbm, zeros_hbm, out_hbm,
                     idx_v, rows_v, acc_sh):
        c = lax.axis_index("c")
        s = lax.axis_index("s")
        wid = c * NUM_SUBCORES + s
        last = NUM_SUBCORES - 1

        @pl.when(s < last)
        def _():
            pltpu.sync_copy(zeros_hbm.at[pl.ds(0, NODE_A)],
                            acc_sh.at[pl.ds(s * NODE_A, NODE_A)])

        @pl.when(s == last)
        def _():
            pltpu.sync_copy(zeros_hbm,
                            acc_sh.at[pl.ds(last * NODE_A, NODE_LAST)])

        pltpu.sync_copy(src_hbm.at[wid], idx_v)
        plsc.subcore_barrier()

        def body(j, carry):
            base = wid * EDGES_PER_TILE + j * CHUNK
            pltpu.sync_copy(msg_hbm.at[pl.ds(base, CHUNK)], rows_v)
            pltpu.sync_copy(rows_v, acc_sh.at[idx_v.at[j]], add=True)
            return carry

        lax.fori_loop(0, NCHUNKS, body, 0)
        plsc.subcore_barrier()

        @pl.when(s < last)
        def _():
            pltpu.sync_copy(acc_sh.at[pl.ds(s * NODE_A, NODE_A)],
                            out_hbm.at[c, pl.ds(s * NODE_A, NODE_A)])

        @pl.when(s == last)
        def _():
            pltpu.sync_copy(acc_sh.at[pl.ds(last * NODE_A, NODE_LAST)],
                            out_hbm.at[c, pl.ds(last * NODE_A, NODE_LAST)])

    return _scatter_sum


NBLK = 2000


def _node_body(hv_ref, p_ref, d1_ref, d1b_ref, d2_ref, d2b_ref,
               g1_ref, bb1_ref, g2_ref, bb2_ref, o_ref):
    dh = (p_ref[0] + p_ref[1]) * SCALE_INV
    h = _ln(hv_ref[...] + dh, g1_ref[...], bb1_ref[...])
    y = jnp.dot(_gelu(jnp.dot(h, d1_ref[...],
                              preferred_element_type=jnp.float32)
                      + d1b_ref[...]),
                d2_ref[...], preferred_element_type=jnp.float32) + d2b_ref[...]
    o_ref[...] = _ln(h + y, g2_ref[...], bb2_ref[...])


def _node_stage(h_V, partials, d1_w, d1_b, d2_w, d2_b,
                ln1_g, ln1_b, ln2_g, ln2_b):
    full = lambda shape: pl.BlockSpec(shape, lambda i: (0, 0))
    return pl.pallas_call(
        _node_body,
        grid=(N_NODES // NBLK,),
        in_specs=[
            pl.BlockSpec((NBLK, H), lambda i: (i, 0)),
            pl.BlockSpec((NUM_CORES, NBLK, H), lambda i: (0, i, 0)),
            full((H, 4 * H)), full((1, 4 * H)),
            full((4 * H, H)), full((1, H)),
            full((1, H)), full((1, H)),
            full((1, H)), full((1, H)),
        ],
        out_specs=pl.BlockSpec((NBLK, H), lambda i: (i, 0)),
        out_shape=jax.ShapeDtypeStruct((N_NODES, H), jnp.float32),
    )(h_V, partials, d1_w, d1_b.reshape(1, 4 * H), d2_w, d2_b.reshape(1, H),
      ln1_g.reshape(1, H), ln1_b.reshape(1, H),
      ln2_g.reshape(1, H), ln2_b.reshape(1, H))


def kernel(h_V, h_E, edge_idx, W1_w, W1_b, W2_w, W2_b, W3_w, W3_b,
           d1_w, d1_b, d2_w, d2_b, ln1_g, ln1_b, ln2_g, ln2_b):
    msg = _edge_mlp(h_E, W1_w, W1_b, W2_w, W2_b, W3_w, W3_b)
    src = edge_idx[0].astype(jnp.int32).reshape(NW, NCHUNKS, CHUNK)
    zeros = jnp.zeros((NODE_LAST, H), jnp.float32)
    partials = _make_scatter_sum()(msg, src, zeros)
    return _node_stage(h_V, partials, d1_w, d1_b, d2_w, d2_b,
                       ln1_g, ln1_b, ln2_g, ln2_b)


# speedup vs baseline: 3.2483x; 1.3328x over previous
import functools

import jax
import jax.numpy as jnp
from jax import lax
from jax.experimental import pallas as pl
from jax.experimental.pallas import tpu as pltpu
from jax.experimental.pallas import tpu_sc as plsc

N_NODES = 10000
N_EDGES = 320000
H = 128
NIN = 16
SCALE_INV = 1.0 / 30.0

NUM_CORES = 2
NUM_SUBCORES = 16
NW = NUM_CORES * NUM_SUBCORES
EDGES_PER_TILE = N_EDGES // NW
CHUNK = 80
NCHUNKS = EDGES_PER_TILE // CHUNK
NODE_A = 624
NODE_LAST = N_NODES - (NUM_SUBCORES - 1) * NODE_A

_SQRT_HALF = 0.7071067811865476


def _erf(x):
    a1 = 0.254829592
    a2 = -0.284496736
    a3 = 1.421413741
    a4 = -1.453152027
    a5 = 1.061405429
    p = 0.3275911
    ax = jnp.abs(x)
    t = 1.0 / (1.0 + p * ax)
    poly = ((((a5 * t + a4) * t + a3) * t + a2) * t + a1) * t
    y = 1.0 - poly * jnp.exp(-ax * ax)
    return jnp.where(x < 0, -y, y)


def _gelu(x):
    return 0.5 * x * (1.0 + _erf(x * _SQRT_HALF))


def _ln(x, g, b, eps=1e-5):
    mu = jnp.mean(x, axis=-1, keepdims=True)
    var = jnp.mean((x - mu) ** 2, axis=-1, keepdims=True)
    return (x - mu) * lax.rsqrt(var + eps) * g + b


EBLK = 3200


def _edge_body(xT_ref, w1_ref, b1_ref, w2_ref, b2_ref, w3_ref, b3_ref, o_ref):
    cT = (((0,), (0,)), ((), ()))
    xT = xT_ref[...]
    m = _gelu(lax.dot_general(w1_ref[...], xT, cT,
                              preferred_element_type=jnp.float32)
              + b1_ref[...])
    m = _gelu(lax.dot_general(w2_ref[...], m, cT,
                              preferred_element_type=jnp.float32)
              + b2_ref[...])
    o_ref[...] = (lax.dot_general(m, w3_ref[...], cT,
                                  preferred_element_type=jnp.float32)
                  + b3_ref[...])


def _edge_mlp(h_E, W1_w, W1_b, W2_w, W2_b, W3_w, W3_b):
    full = lambda shape: pl.BlockSpec(shape, lambda i: (0, 0))
    return pl.pallas_call(
        _edge_body,
        grid=(N_EDGES // EBLK,),
        in_specs=[
            pl.BlockSpec((H + NIN, EBLK), lambda i: (0, i)),
            full((H + NIN, H)), full((H, 1)),
            full((H, H)), full((H, 1)),
            full((H, H)), full((1, H)),
        ],
        out_specs=pl.BlockSpec((EBLK, H), lambda i: (i, 0)),
        out_shape=jax.ShapeDtypeStruct((N_EDGES, H), jnp.float32),
    )(h_E.T, W1_w, W1_b.reshape(H, 1), W2_w, W2_b.reshape(H, 1),
      W3_w, W3_b.reshape(1, H))


@functools.lru_cache(maxsize=1)
def _make_scatter_sum():
    mesh = plsc.VectorSubcoreMesh(core_axis_name="c", subcore_axis_name="s")

    @functools.partial(
        pl.kernel,
        mesh=mesh,
        out_type=jax.ShapeDtypeStruct((NUM_CORES, N_NODES, H), jnp.float32),
        scratch_types=[
            pltpu.VMEM((NCHUNKS, CHUNK), jnp.int32),
            pltpu.VMEM((CHUNK, H), jnp.float32),
            pltpu.VMEM((CHUNK, H), jnp.float32),
            pltpu.VMEM_SHARED((N_NODES, H), jnp.float32),
            pltpu.SemaphoreType.DMA,
            pltpu.SemaphoreType.DMA,
        ],
    )
    def _scatter_sum(msg_hbm, src_hbm, zeros_hbm, out_hbm,
                     idx_v, rows0_v, rows1_v, acc_sh, sem0, sem1):
        c = lax.axis_index("c")
        s = lax.axis_index("s")
        wid = c * NUM_SUBCORES + s
        last = NUM_SUBCORES - 1

        @pl.when(s < last)
        def _():
            pltpu.sync_copy(zeros_hbm.at[pl.ds(0, NODE_A)],
                            acc_sh.at[pl.ds(s * NODE_A, NODE_A)])

        @pl.when(s == last)
        def _():
            pltpu.sync_copy(zeros_hbm,
                            acc_sh.at[pl.ds(last * NODE_A, NODE_LAST)])

        pltpu.sync_copy(src_hbm.at[wid], idx_v)
        plsc.subcore_barrier()

        ebase = wid * EDGES_PER_TILE

        def _gather(j, buf, sem):
            return pltpu.async_copy(
                msg_hbm.at[pl.ds(ebase + j * CHUNK, CHUNK)], buf, sem)

        _gather(0, rows0_v, sem0)

        def pair_body(p, carry):
            j0 = 2 * p
            pltpu.make_async_copy(
                msg_hbm.at[pl.ds(ebase + j0 * CHUNK, CHUNK)],
                rows0_v, sem0).wait()
            _gather(j0 + 1, rows1_v, sem1)
            pltpu.sync_copy(rows0_v, acc_sh.at[idx_v.at[j0]], add=True)
            pltpu.make_async_copy(
                msg_hbm.at[pl.ds(ebase + (j0 + 1) * CHUNK, CHUNK)],
                rows1_v, sem1).wait()
            _gather(j0 + 2, rows0_v, sem0)
            pltpu.sync_copy(rows1_v, acc_sh.at[idx_v.at[j0 + 1]], add=True)
            return carry

        lax.fori_loop(0, (NCHUNKS - 1) // 2, pair_body, 0)
        last_j = NCHUNKS - 1
        pltpu.make_async_copy(
            msg_hbm.at[pl.ds(ebase + last_j * CHUNK, CHUNK)],
            rows0_v, sem0).wait()
        pltpu.sync_copy(rows0_v, acc_sh.at[idx_v.at[last_j]], add=True)
        plsc.subcore_barrier()

        @pl.when(s < last)
        def _():
            pltpu.sync_copy(acc_sh.at[pl.ds(s * NODE_A, NODE_A)],
                            out_hbm.at[c, pl.ds(s * NODE_A, NODE_A)])

        @pl.when(s == last)
        def _():
            pltpu.sync_copy(acc_sh.at[pl.ds(last * NODE_A, NODE_LAST)],
                            out_hbm.at[c, pl.ds(last * NODE_A, NODE_LAST)])

    return _scatter_sum


NBLK = 2000


def _node_body(hv_ref, p_ref, d1_ref, d1b_ref, d2_ref, d2b_ref,
               g1_ref, bb1_ref, g2_ref, bb2_ref, o_ref):
    dh = (p_ref[0] + p_ref[1]) * SCALE_INV
    h = _ln(hv_ref[...] + dh, g1_ref[...], bb1_ref[...])
    y = jnp.dot(_gelu(jnp.dot(h, d1_ref[...],
                              preferred_element_type=jnp.float32)
                      + d1b_ref[...]),
                d2_ref[...], preferred_element_type=jnp.float32) + d2b_ref[...]
    o_ref[...] = _ln(h + y, g2_ref[...], bb2_ref[...])


def _node_stage(h_V, partials, d1_w, d1_b, d2_w, d2_b,
                ln1_g, ln1_b, ln2_g, ln2_b):
    full = lambda shape: pl.BlockSpec(shape, lambda i: (0, 0))
    return pl.pallas_call(
        _node_body,
        grid=(N_NODES // NBLK,),
        in_specs=[
            pl.BlockSpec((NBLK, H), lambda i: (i, 0)),
            pl.BlockSpec((NUM_CORES, NBLK, H), lambda i: (0, i, 0)),
            full((H, 4 * H)), full((1, 4 * H)),
            full((4 * H, H)), full((1, H)),
            full((1, H)), full((1, H)),
            full((1, H)), full((1, H)),
        ],
        out_specs=pl.BlockSpec((NBLK, H), lambda i: (i, 0)),
        out_shape=jax.ShapeDtypeStruct((N_NODES, H), jnp.float32),
    )(h_V, partials, d1_w, d1_b.reshape(1, 4 * H), d2_w, d2_b.reshape(1, H),
      ln1_g.reshape(1, H), ln1_b.reshape(1, H),
      ln2_g.reshape(1, H), ln2_b.reshape(1, H))


def kernel(h_V, h_E, edge_idx, W1_w, W1_b, W2_w, W2_b, W3_w, W3_b,
           d1_w, d1_b, d2_w, d2_b, ln1_g, ln1_b, ln2_g, ln2_b):
    msg = _edge_mlp(h_E, W1_w, W1_b, W2_w, W2_b, W3_w, W3_b)
    src = edge_idx[0].astype(jnp.int32).reshape(NW, NCHUNKS, CHUNK)
    zeros = jnp.zeros((NODE_LAST, H), jnp.float32)
    partials = _make_scatter_sum()(msg, src, zeros)
    return _node_stage(h_V, partials, d1_w, d1_b, d2_w, d2_b,
                       ln1_g, ln1_b, ln2_g, ln2_b)


# speedup vs baseline: 5.1429x; 1.5833x over previous
import functools

import jax
import jax.numpy as jnp
from jax import lax
from jax.experimental import pallas as pl
from jax.experimental.pallas import tpu as pltpu
from jax.experimental.pallas import tpu_sc as plsc

N_NODES = 10000
N_EDGES = 320000
H = 128
NIN = 16
SCALE_INV = 1.0 / 30.0

NUM_CORES = 2
NUM_SUBCORES = 16
NW = NUM_CORES * NUM_SUBCORES
EDGES_PER_TILE = N_EDGES // NW
CHUNK = 80
NCHUNKS = EDGES_PER_TILE // CHUNK
NODE_A = 624
NODE_LAST = N_NODES - (NUM_SUBCORES - 1) * NODE_A

_SQRT_HALF = 0.7071067811865476


def _erf(x):
    a1 = 0.254829592
    a2 = -0.284496736
    a3 = 1.421413741
    a4 = -1.453152027
    a5 = 1.061405429
    p = 0.3275911
    ax = jnp.abs(x)
    t = 1.0 / (1.0 + p * ax)
    poly = ((((a5 * t + a4) * t + a3) * t + a2) * t + a1) * t
    y = 1.0 - poly * jnp.exp(-ax * ax)
    return jnp.where(x < 0, -y, y)


def _gelu(x):
    return 0.5 * x * (1.0 + lax.erf(x * _SQRT_HALF))


def _ln(x, g, b, eps=1e-5):
    mu = jnp.mean(x, axis=-1, keepdims=True)
    var = jnp.mean((x - mu) ** 2, axis=-1, keepdims=True)
    return (x - mu) * lax.rsqrt(var + eps) * g + b


EBLK = 3200


def _edge_body(xT_ref, w1_ref, b1_ref, w2_ref, b2_ref, w3_ref, b3_ref, o_ref):
    cT = (((0,), (0,)), ((), ()))
    xT = xT_ref[...]
    m = _gelu(lax.dot_general(w1_ref[...], xT, cT,
                              preferred_element_type=jnp.float32)
              + b1_ref[...])
    m = _gelu(lax.dot_general(w2_ref[...], m, cT,
                              preferred_element_type=jnp.float32)
              + b2_ref[...])
    o_ref[...] = (lax.dot_general(m, w3_ref[...], cT,
                                  preferred_element_type=jnp.float32)
                  + b3_ref[...])


def _edge_mlp(h_E, W1_w, W1_b, W2_w, W2_b, W3_w, W3_b):
    full = lambda shape: pl.BlockSpec(shape, lambda i: (0, 0))
    return pl.pallas_call(
        _edge_body,
        grid=(N_EDGES // EBLK,),
        in_specs=[
            pl.BlockSpec((H + NIN, EBLK), lambda i: (0, i)),
            full((H + NIN, H)), full((H, 1)),
            full((H, H)), full((H, 1)),
            full((H, H)), full((1, H)),
        ],
        out_specs=pl.BlockSpec((EBLK, H), lambda i: (i, 0)),
        out_shape=jax.ShapeDtypeStruct((N_EDGES, H), jnp.float32),
    )(h_E.T, W1_w, W1_b.reshape(H, 1), W2_w, W2_b.reshape(H, 1),
      W3_w, W3_b.reshape(1, H))


@functools.lru_cache(maxsize=1)
def _make_scatter_sum():
    mesh = plsc.VectorSubcoreMesh(core_axis_name="c", subcore_axis_name="s")

    @functools.partial(
        pl.kernel,
        mesh=mesh,
        out_type=jax.ShapeDtypeStruct((NUM_CORES, N_NODES, H), jnp.float32),
        scratch_types=[
            pltpu.VMEM((NCHUNKS, CHUNK), jnp.int32),
            pltpu.VMEM((CHUNK, H), jnp.float32),
            pltpu.VMEM((CHUNK, H), jnp.float32),
            pltpu.VMEM_SHARED((N_NODES, H), jnp.float32),
            pltpu.SemaphoreType.DMA,
            pltpu.SemaphoreType.DMA,
        ],
    )
    def _scatter_sum(msg_hbm, src_hbm, zeros_hbm, out_hbm,
                     idx_v, rows0_v, rows1_v, acc_sh, sem0, sem1):
        c = lax.axis_index("c")
        s = lax.axis_index("s")
        wid = c * NUM_SUBCORES + s
        last = NUM_SUBCORES - 1

        @pl.when(s < last)
        def _():
            pltpu.sync_copy(zeros_hbm.at[pl.ds(0, NODE_A)],
                            acc_sh.at[pl.ds(s * NODE_A, NODE_A)])

        @pl.when(s == last)
        def _():
            pltpu.sync_copy(zeros_hbm,
                            acc_sh.at[pl.ds(last * NODE_A, NODE_LAST)])

        pltpu.sync_copy(src_hbm.at[wid], idx_v)
        plsc.subcore_barrier()

        ebase = wid * EDGES_PER_TILE

        def _gather(j, buf, sem):
            return pltpu.async_copy(
                msg_hbm.at[pl.ds(ebase + j * CHUNK, CHUNK)], buf, sem)

        _gather(0, rows0_v, sem0)

        def pair_body(p, carry):
            j0 = 2 * p
            pltpu.make_async_copy(
                msg_hbm.at[pl.ds(ebase + j0 * CHUNK, CHUNK)],
                rows0_v, sem0).wait()
            _gather(j0 + 1, rows1_v, sem1)
            pltpu.sync_copy(rows0_v, acc_sh.at[idx_v.at[j0]], add=True)
            pltpu.make_async_copy(
                msg_hbm.at[pl.ds(ebase + (j0 + 1) * CHUNK, CHUNK)],
                rows1_v, sem1).wait()
            _gather(j0 + 2, rows0_v, sem0)
            pltpu.sync_copy(rows1_v, acc_sh.at[idx_v.at[j0 + 1]], add=True)
            return carry

        lax.fori_loop(0, (NCHUNKS - 1) // 2, pair_body, 0)
        last_j = NCHUNKS - 1
        pltpu.make_async_copy(
            msg_hbm.at[pl.ds(ebase + last_j * CHUNK, CHUNK)],
            rows0_v, sem0).wait()
        pltpu.sync_copy(rows0_v, acc_sh.at[idx_v.at[last_j]], add=True)
        plsc.subcore_barrier()

        @pl.when(s < last)
        def _():
            pltpu.sync_copy(acc_sh.at[pl.ds(s * NODE_A, NODE_A)],
                            out_hbm.at[c, pl.ds(s * NODE_A, NODE_A)])

        @pl.when(s == last)
        def _():
            pltpu.sync_copy(acc_sh.at[pl.ds(last * NODE_A, NODE_LAST)],
                            out_hbm.at[c, pl.ds(last * NODE_A, NODE_LAST)])

    return _scatter_sum


NBLK = 2000


def _node_body(hv_ref, p_ref, d1_ref, d1b_ref, d2_ref, d2b_ref,
               g1_ref, bb1_ref, g2_ref, bb2_ref, o_ref):
    dh = (p_ref[0] + p_ref[1]) * SCALE_INV
    h = _ln(hv_ref[...] + dh, g1_ref[...], bb1_ref[...])
    y = jnp.dot(_gelu(jnp.dot(h, d1_ref[...],
                              preferred_element_type=jnp.float32)
                      + d1b_ref[...]),
                d2_ref[...], preferred_element_type=jnp.float32) + d2b_ref[...]
    o_ref[...] = _ln(h + y, g2_ref[...], bb2_ref[...])


def _node_stage(h_V, partials, d1_w, d1_b, d2_w, d2_b,
                ln1_g, ln1_b, ln2_g, ln2_b):
    full = lambda shape: pl.BlockSpec(shape, lambda i: (0, 0))
    return pl.pallas_call(
        _node_body,
        grid=(N_NODES // NBLK,),
        in_specs=[
            pl.BlockSpec((NBLK, H), lambda i: (i, 0)),
            pl.BlockSpec((NUM_CORES, NBLK, H), lambda i: (0, i, 0)),
            full((H, 4 * H)), full((1, 4 * H)),
            full((4 * H, H)), full((1, H)),
            full((1, H)), full((1, H)),
            full((1, H)), full((1, H)),
        ],
        out_specs=pl.BlockSpec((NBLK, H), lambda i: (i, 0)),
        out_shape=jax.ShapeDtypeStruct((N_NODES, H), jnp.float32),
    )(h_V, partials, d1_w, d1_b.reshape(1, 4 * H), d2_w, d2_b.reshape(1, H),
      ln1_g.reshape(1, H), ln1_b.reshape(1, H),
      ln2_g.reshape(1, H), ln2_b.reshape(1, H))


def kernel(h_V, h_E, edge_idx, W1_w, W1_b, W2_w, W2_b, W3_w, W3_b,
           d1_w, d1_b, d2_w, d2_b, ln1_g, ln1_b, ln2_g, ln2_b):
    msg = _edge_mlp(h_E, W1_w, W1_b, W2_w, W2_b, W3_w, W3_b)
    src = edge_idx[0].astype(jnp.int32).reshape(NW, NCHUNKS, CHUNK)
    zeros = jnp.zeros((NODE_LAST, H), jnp.float32)
    partials = _make_scatter_sum()(msg, src, zeros)
    return _node_stage(h_V, partials, d1_w, d1_b, d2_w, d2_b,
                       ln1_g, ln1_b, ln2_g, ln2_b)


# speedup vs baseline: 5.1496x; 1.0013x over previous
import functools

import jax
import jax.numpy as jnp
from jax import lax
from jax.experimental import pallas as pl
from jax.experimental.pallas import tpu as pltpu
from jax.experimental.pallas import tpu_sc as plsc

N_NODES = 10000
N_EDGES = 320000
H = 128
NIN = 16
SCALE_INV = 1.0 / 30.0

NUM_CORES = 2
NUM_SUBCORES = 16
NW = NUM_CORES * NUM_SUBCORES
EDGES_PER_TILE = N_EDGES // NW
CHUNK = 80
NCHUNKS = EDGES_PER_TILE // CHUNK
NODE_A = 624
NODE_LAST = N_NODES - (NUM_SUBCORES - 1) * NODE_A

_SQRT_HALF = 0.7071067811865476


def _erf(x):
    a1 = 0.254829592
    a2 = -0.284496736
    a3 = 1.421413741
    a4 = -1.453152027
    a5 = 1.061405429
    p = 0.3275911
    ax = jnp.abs(x)
    t = 1.0 / (1.0 + p * ax)
    poly = ((((a5 * t + a4) * t + a3) * t + a2) * t + a1) * t
    y = 1.0 - poly * jnp.exp(-ax * ax)
    return jnp.where(x < 0, -y, y)


def _gelu(x):
    return 0.5 * x * (1.0 + lax.erf(x * _SQRT_HALF))


def _ln(x, g, b, eps=1e-5):
    mu = jnp.mean(x, axis=-1, keepdims=True)
    var = jnp.mean((x - mu) ** 2, axis=-1, keepdims=True)
    return (x - mu) * lax.rsqrt(var + eps) * g + b


EBLK = 3200


def _edge_body(xT_ref, w1_ref, b1_ref, w2_ref, b2_ref, w3_ref, b3_ref, o_ref):
    cT = (((0,), (0,)), ((), ()))
    xT = xT_ref[...]
    m = _gelu(lax.dot_general(w1_ref[...], xT, cT,
                              preferred_element_type=jnp.float32)
              + b1_ref[...])
    m = _gelu(lax.dot_general(w2_ref[...], m, cT,
                              preferred_element_type=jnp.float32)
              + b2_ref[...])
    o_ref[...] = (lax.dot_general(m, w3_ref[...], cT,
                                  preferred_element_type=jnp.float32)
                  + b3_ref[...])


def _edge_mlp(h_E, W1_w, W1_b, W2_w, W2_b, W3_w, W3_b):
    full = lambda shape: pl.BlockSpec(shape, lambda i: (0, 0))
    return pl.pallas_call(
        _edge_body,
        grid=(N_EDGES // EBLK,),
        in_specs=[
            pl.BlockSpec((H + NIN, EBLK), lambda i: (0, i)),
            full((H + NIN, H)), full((H, 1)),
            full((H, H)), full((H, 1)),
            full((H, H)), full((1, H)),
        ],
        out_specs=pl.BlockSpec((EBLK, H), lambda i: (i, 0)),
        out_shape=jax.ShapeDtypeStruct((N_EDGES, H), jnp.float32),
    )(h_E.T, W1_w, W1_b.reshape(H, 1), W2_w, W2_b.reshape(H, 1),
      W3_w, W3_b.reshape(1, H))


@functools.lru_cache(maxsize=1)
def _make_scatter_sum():
    mesh = plsc.VectorSubcoreMesh(core_axis_name="c", subcore_axis_name="s")

    @functools.partial(
        pl.kernel,
        mesh=mesh,
        out_type=jax.ShapeDtypeStruct((NUM_CORES, N_NODES, H), jnp.float32),
        scratch_types=[
            pltpu.VMEM((NCHUNKS, CHUNK), jnp.int32),
            pltpu.VMEM((CHUNK, H), jnp.float32),
            pltpu.VMEM((CHUNK, H), jnp.float32),
            pltpu.VMEM_SHARED((N_NODES, H), jnp.float32),
            pltpu.SemaphoreType.DMA,
            pltpu.SemaphoreType.DMA,
            pltpu.SemaphoreType.DMA,
            pltpu.SemaphoreType.DMA,
        ],
    )
    def _scatter_sum(msg_hbm, src_hbm, zeros_hbm, out_hbm,
                     idx_v, rows0_v, rows1_v, acc_sh,
                     sem_g0, sem_g1, sem_s0, sem_s1):
        c = lax.axis_index("c")
        s = lax.axis_index("s")
        wid = c * NUM_SUBCORES + s
        last = NUM_SUBCORES - 1

        @pl.when(s < last)
        def _():
            pltpu.sync_copy(zeros_hbm.at[pl.ds(0, NODE_A)],
                            acc_sh.at[pl.ds(s * NODE_A, NODE_A)])

        @pl.when(s == last)
        def _():
            pltpu.sync_copy(zeros_hbm,
                            acc_sh.at[pl.ds(last * NODE_A, NODE_LAST)])

        pltpu.sync_copy(src_hbm.at[wid], idx_v)
        plsc.subcore_barrier()

        ebase = wid * EDGES_PER_TILE

        def _g(j, buf, sem):
            pltpu.async_copy(
                msg_hbm.at[pl.ds(ebase + j * CHUNK, CHUNK)], buf, sem)

        def _wg(j, buf, sem):
            pltpu.make_async_copy(
                msg_hbm.at[pl.ds(ebase + j * CHUNK, CHUNK)], buf, sem).wait()

        def _s(j, buf, sem):
            pltpu.async_copy(buf, acc_sh.at[idx_v.at[j]], sem, add=True)

        def _ws(buf, sem):
            pltpu.make_async_copy(buf, acc_sh.at[idx_v.at[0]], sem).wait()

        _g(0, rows0_v, sem_g0)
        _wg(0, rows0_v, sem_g0)
        _s(0, rows0_v, sem_s0)
        _g(1, rows1_v, sem_g1)

        def body(p, carry):
            j = 2 * p
            _wg(j + 1, rows1_v, sem_g1)
            _s(j + 1, rows1_v, sem_s1)
            _ws(rows0_v, sem_s0)
            _g(j + 2, rows0_v, sem_g0)
            _wg(j + 2, rows0_v, sem_g0)
            _s(j + 2, rows0_v, sem_s0)
            _ws(rows1_v, sem_s1)
            _g(j + 3, rows1_v, sem_g1)
            return carry

        lax.fori_loop(0, (NCHUNKS - 3) // 2, body, 0)
        _wg(NCHUNKS - 2, rows1_v, sem_g1)
        _s(NCHUNKS - 2, rows1_v, sem_s1)
        _ws(rows0_v, sem_s0)
        _g(NCHUNKS - 1, rows0_v, sem_g0)
        _wg(NCHUNKS - 1, rows0_v, sem_g0)
        _s(NCHUNKS - 1, rows0_v, sem_s0)
        _ws(rows1_v, sem_s1)
        _ws(rows0_v, sem_s0)
        plsc.subcore_barrier()

        @pl.when(s < last)
        def _():
            pltpu.sync_copy(acc_sh.at[pl.ds(s * NODE_A, NODE_A)],
                            out_hbm.at[c, pl.ds(s * NODE_A, NODE_A)])

        @pl.when(s == last)
        def _():
            pltpu.sync_copy(acc_sh.at[pl.ds(last * NODE_A, NODE_LAST)],
                            out_hbm.at[c, pl.ds(last * NODE_A, NODE_LAST)])

    return _scatter_sum


NBLK = 2000


def _node_body(hv_ref, p_ref, d1_ref, d1b_ref, d2_ref, d2b_ref,
               g1_ref, bb1_ref, g2_ref, bb2_ref, o_ref):
    dh = (p_ref[0] + p_ref[1]) * SCALE_INV
    h = _ln(hv_ref[...] + dh, g1_ref[...], bb1_ref[...])
    y = jnp.dot(_gelu(jnp.dot(h, d1_ref[...],
                              preferred_element_type=jnp.float32)
                      + d1b_ref[...]),
                d2_ref[...], preferred_element_type=jnp.float32) + d2b_ref[...]
    o_ref[...] = _ln(h + y, g2_ref[...], bb2_ref[...])


def _node_stage(h_V, partials, d1_w, d1_b, d2_w, d2_b,
                ln1_g, ln1_b, ln2_g, ln2_b):
    full = lambda shape: pl.BlockSpec(shape, lambda i: (0, 0))
    return pl.pallas_call(
        _node_body,
        grid=(N_NODES // NBLK,),
        in_specs=[
            pl.BlockSpec((NBLK, H), lambda i: (i, 0)),
            pl.BlockSpec((NUM_CORES, NBLK, H), lambda i: (0, i, 0)),
            full((H, 4 * H)), full((1, 4 * H)),
            full((4 * H, H)), full((1, H)),
            full((1, H)), full((1, H)),
            full((1, H)), full((1, H)),
        ],
        out_specs=pl.BlockSpec((NBLK, H), lambda i: (i, 0)),
        out_shape=jax.ShapeDtypeStruct((N_NODES, H), jnp.float32),
    )(h_V, partials, d1_w, d1_b.reshape(1, 4 * H), d2_w, d2_b.reshape(1, H),
      ln1_g.reshape(1, H), ln1_b.reshape(1, H),
      ln2_g.reshape(1, H), ln2_b.reshape(1, H))


def kernel(h_V, h_E, edge_idx, W1_w, W1_b, W2_w, W2_b, W3_w, W3_b,
           d1_w, d1_b, d2_w, d2_b, ln1_g, ln1_b, ln2_g, ln2_b):
    msg = _edge_mlp(h_E, W1_w, W1_b, W2_w, W2_b, W3_w, W3_b)
    src = edge_idx[0].astype(jnp.int32).reshape(NW, NCHUNKS, CHUNK)
    zeros = jnp.zeros((NODE_LAST, H), jnp.float32)
    partials = _make_scatter_sum()(msg, src, zeros)
    return _node_stage(h_V, partials, d1_w, d1_b, d2_w, d2_b,
                       ln1_g, ln1_b, ln2_g, ln2_b)


# speedup vs baseline: 5.8989x; 1.1455x over previous
import functools

import jax
import jax.numpy as jnp
from jax import lax
from jax.experimental import pallas as pl
from jax.experimental.pallas import tpu as pltpu
from jax.experimental.pallas import tpu_sc as plsc

N_NODES = 10000
N_EDGES = 320000
H = 128
NIN = 16
SCALE_INV = 1.0 / 30.0

NUM_CORES = 2
NUM_SUBCORES = 16
NW = NUM_CORES * NUM_SUBCORES
EDGES_PER_TILE = N_EDGES // NW
CHUNK = 80
KA = 65
KB = 60
NODE_A = 624
NODE_LAST = N_NODES - (NUM_SUBCORES - 1) * NODE_A

_SQRT_HALF = 0.7071067811865476


def _erf(x):
    a1 = 0.254829592
    a2 = -0.284496736
    a3 = 1.421413741
    a4 = -1.453152027
    a5 = 1.061405429
    p = 0.3275911
    ax = jnp.abs(x)
    t = 1.0 / (1.0 + p * ax)
    poly = ((((a5 * t + a4) * t + a3) * t + a2) * t + a1) * t
    y = 1.0 - poly * jnp.exp(-ax * ax)
    return jnp.where(x < 0, -y, y)


def _gelu(x):
    return 0.5 * x * (1.0 + lax.erf(x * _SQRT_HALF))


def _ln(x, g, b, eps=1e-5):
    mu = jnp.mean(x, axis=-1, keepdims=True)
    var = jnp.mean((x - mu) ** 2, axis=-1, keepdims=True)
    return (x - mu) * lax.rsqrt(var + eps) * g + b


EBLK = 3200


def _edge_body(xT_ref, w1_ref, b1_ref, w2_ref, b2_ref, w3_ref, b3_ref, o_ref):
    cT = (((0,), (0,)), ((), ()))
    xT = xT_ref[...]
    m = _gelu(lax.dot_general(w1_ref[...], xT, cT,
                              preferred_element_type=jnp.float32)
              + b1_ref[...])
    m = _gelu(lax.dot_general(w2_ref[...], m, cT,
                              preferred_element_type=jnp.float32)
              + b2_ref[...])
    o_ref[...] = (lax.dot_general(m, w3_ref[...], cT,
                                  preferred_element_type=jnp.float32)
                  + b3_ref[...])


def _edge_mlp(h_E_T, W1_w, W1_b, W2_w, W2_b, W3_w, W3_b,
              n_edges, block_offset):
    full = lambda shape: pl.BlockSpec(shape, lambda i: (0, 0))
    return pl.pallas_call(
        _edge_body,
        grid=(n_edges // EBLK,),
        in_specs=[
            pl.BlockSpec((H + NIN, EBLK), lambda i: (0, i + block_offset)),
            full((H + NIN, H)), full((H, 1)),
            full((H, H)), full((H, 1)),
            full((H, H)), full((1, H)),
        ],
        out_specs=pl.BlockSpec((EBLK, H), lambda i: (i, 0)),
        out_shape=jax.ShapeDtypeStruct((n_edges, H), jnp.float32),
    )(h_E_T, W1_w, W1_b.reshape(H, 1), W2_w, W2_b.reshape(H, 1),
      W3_w, W3_b.reshape(1, H))


@functools.lru_cache(maxsize=2)
def _make_scatter_sum(nchunks):
    mesh = plsc.VectorSubcoreMesh(core_axis_name="c", subcore_axis_name="s")
    ept = nchunks * CHUNK

    @functools.partial(
        pl.kernel,
        mesh=mesh,
        out_type=jax.ShapeDtypeStruct((NUM_CORES, N_NODES, H), jnp.float32),
        scratch_types=[
            pltpu.VMEM((nchunks, CHUNK), jnp.int32),
            pltpu.VMEM((CHUNK, H), jnp.float32),
            pltpu.VMEM((CHUNK, H), jnp.float32),
            pltpu.VMEM_SHARED((N_NODES, H), jnp.float32),
            pltpu.SemaphoreType.DMA,
            pltpu.SemaphoreType.DMA,
            pltpu.SemaphoreType.DMA,
            pltpu.SemaphoreType.DMA,
        ],
    )
    def _scatter_sum(msg_hbm, src_hbm, init_hbm, out_hbm,
                     idx_v, rows0_v, rows1_v, acc_sh,
                     sem_g0, sem_g1, sem_s0, sem_s1):
        c = lax.axis_index("c")
        s = lax.axis_index("s")
        wid = c * NUM_SUBCORES + s
        last = NUM_SUBCORES - 1

        @pl.when(s < last)
        def _():
            pltpu.sync_copy(init_hbm.at[c, pl.ds(s * NODE_A, NODE_A)],
                            acc_sh.at[pl.ds(s * NODE_A, NODE_A)])

        @pl.when(s == last)
        def _():
            pltpu.sync_copy(init_hbm.at[c, pl.ds(last * NODE_A, NODE_LAST)],
                            acc_sh.at[pl.ds(last * NODE_A, NODE_LAST)])

        pltpu.sync_copy(src_hbm.at[wid], idx_v)
        plsc.subcore_barrier()

        ebase = wid * ept

        def _g(j, buf, sem):
            pltpu.async_copy(
                msg_hbm.at[pl.ds(ebase + j * CHUNK, CHUNK)], buf, sem)

        def _wg(j, buf, sem):
            pltpu.make_async_copy(
                msg_hbm.at[pl.ds(ebase + j * CHUNK, CHUNK)], buf,
                sem).wait()

        def _s(j, buf, sem):
            pltpu.async_copy(buf, acc_sh.at[idx_v.at[j]], sem, add=True)

        def _ws(buf, sem):
            pltpu.make_async_copy(buf, acc_sh.at[idx_v.at[0]], sem).wait()

        _g(0, rows0_v, sem_g0)
        _wg(0, rows0_v, sem_g0)
        _s(0, rows0_v, sem_s0)
        _g(1, rows1_v, sem_g1)

        def body(p, carry):
            j = 2 * p
            _wg(j + 1, rows1_v, sem_g1)
            _s(j + 1, rows1_v, sem_s1)
            _ws(rows0_v, sem_s0)
            _g(j + 2, rows0_v, sem_g0)
            _wg(j + 2, rows0_v, sem_g0)
            _s(j + 2, rows0_v, sem_s0)
            _ws(rows1_v, sem_s1)
            _g(j + 3, rows1_v, sem_g1)
            return carry

        if nchunks % 2:
            lax.fori_loop(0, (nchunks - 3) // 2, body, 0)
            _wg(nchunks - 2, rows1_v, sem_g1)
            _s(nchunks - 2, rows1_v, sem_s1)
            _ws(rows0_v, sem_s0)
            _g(nchunks - 1, rows0_v, sem_g0)
            _wg(nchunks - 1, rows0_v, sem_g0)
            _s(nchunks - 1, rows0_v, sem_s0)
            _ws(rows1_v, sem_s1)
            _ws(rows0_v, sem_s0)
        else:
            lax.fori_loop(0, (nchunks - 2) // 2, body, 0)
            _wg(nchunks - 1, rows1_v, sem_g1)
            _s(nchunks - 1, rows1_v, sem_s1)
            _ws(rows0_v, sem_s0)
            _ws(rows1_v, sem_s1)
        plsc.subcore_barrier()

        @pl.when(s < last)
        def _():
            pltpu.sync_copy(acc_sh.at[pl.ds(s * NODE_A, NODE_A)],
                            out_hbm.at[c, pl.ds(s * NODE_A, NODE_A)])

        @pl.when(s == last)
        def _():
            pltpu.sync_copy(acc_sh.at[pl.ds(last * NODE_A, NODE_LAST)],
                            out_hbm.at[c, pl.ds(last * NODE_A, NODE_LAST)])

    return _scatter_sum


NBLK = 2000


def _node_body(hv_ref, p_ref, d1_ref, d1b_ref, d2_ref, d2b_ref,
               g1_ref, bb1_ref, g2_ref, bb2_ref, o_ref):
    dh = (p_ref[0] + p_ref[1]) * SCALE_INV
    h = _ln(hv_ref[...] + dh, g1_ref[...], bb1_ref[...])
    y = jnp.dot(_gelu(jnp.dot(h, d1_ref[...],
                              preferred_element_type=jnp.float32)
                      + d1b_ref[...]),
                d2_ref[...], preferred_element_type=jnp.float32) + d2b_ref[...]
    o_ref[...] = _ln(h + y, g2_ref[...], bb2_ref[...])


def _node_stage(h_V, partials, d1_w, d1_b, d2_w, d2_b,
                ln1_g, ln1_b, ln2_g, ln2_b):
    full = lambda shape: pl.BlockSpec(shape, lambda i: (0, 0))
    return pl.pallas_call(
        _node_body,
        grid=(N_NODES // NBLK,),
        in_specs=[
            pl.BlockSpec((NBLK, H), lambda i: (i, 0)),
            pl.BlockSpec((NUM_CORES, NBLK, H), lambda i: (0, i, 0)),
            full((H, 4 * H)), full((1, 4 * H)),
            full((4 * H, H)), full((1, H)),
            full((1, H)), full((1, H)),
            full((1, H)), full((1, H)),
        ],
        out_specs=pl.BlockSpec((NBLK, H), lambda i: (i, 0)),
        out_shape=jax.ShapeDtypeStruct((N_NODES, H), jnp.float32),
    )(h_V, partials, d1_w, d1_b.reshape(1, 4 * H), d2_w, d2_b.reshape(1, H),
      ln1_g.reshape(1, H), ln1_b.reshape(1, H),
      ln2_g.reshape(1, H), ln2_b.reshape(1, H))


def kernel(h_V, h_E, edge_idx, W1_w, W1_b, W2_w, W2_b, W3_w, W3_b,
           d1_w, d1_b, d2_w, d2_b, ln1_g, ln1_b, ln2_g, ln2_b):
    e_a = NW * KA * CHUNK
    h_E_T = h_E.T
    mlp_w = (W1_w, W1_b, W2_w, W2_b, W3_w, W3_b)
    msg_a = _edge_mlp(h_E_T, *mlp_w, e_a, 0)
    msg_b = _edge_mlp(h_E_T, *mlp_w, N_EDGES - e_a, e_a // EBLK)
    src = edge_idx[0].astype(jnp.int32)
    src_a = src[:e_a].reshape(NW, KA, CHUNK)
    src_b = src[e_a:].reshape(NW, KB, CHUNK)
    init0 = jnp.zeros((NUM_CORES, N_NODES, H), jnp.float32)
    part_a = _make_scatter_sum(KA)(msg_a, src_a, init0)
    part_b = _make_scatter_sum(KB)(msg_b, src_b, part_a)
    return _node_stage(h_V, part_b, d1_w, d1_b, d2_w, d2_b,
                       ln1_g, ln1_b, ln2_g, ln2_b)


# speedup vs baseline: 6.1536x; 1.0432x over previous
import functools

import jax
import jax.numpy as jnp
from jax import lax
from jax.experimental import pallas as pl
from jax.experimental.pallas import tpu as pltpu
from jax.experimental.pallas import tpu_sc as plsc

N_NODES = 10000
N_EDGES = 320000
H = 128
NIN = 16
SCALE_INV = 1.0 / 30.0

NUM_CORES = 2
NUM_SUBCORES = 16
NW = NUM_CORES * NUM_SUBCORES
EDGES_PER_TILE = N_EDGES // NW
CHUNK = 80
KA = 65
KB = 60
NODE_A = 624
NODE_LAST = N_NODES - (NUM_SUBCORES - 1) * NODE_A

_SQRT_HALF = 0.7071067811865476


def _erf(x):
    a1 = 0.254829592
    a2 = -0.284496736
    a3 = 1.421413741
    a4 = -1.453152027
    a5 = 1.061405429
    p = 0.3275911
    ax = jnp.abs(x)
    t = 1.0 / (1.0 + p * ax)
    poly = ((((a5 * t + a4) * t + a3) * t + a2) * t + a1) * t
    y = 1.0 - poly * jnp.exp(-ax * ax)
    return jnp.where(x < 0, -y, y)


def _gelu(x):
    return 0.5 * x * (1.0 + lax.erf(x * _SQRT_HALF))


def _ln(x, g, b, eps=1e-5):
    mu = jnp.mean(x, axis=-1, keepdims=True)
    var = jnp.mean((x - mu) ** 2, axis=-1, keepdims=True)
    return (x - mu) * lax.rsqrt(var + eps) * g + b


EBLK = 3200


def _edge_body(xT_ref, w1_ref, b1_ref, w2_ref, b2_ref, w3_ref, b3_ref, o_ref):
    cT = (((0,), (0,)), ((), ()))
    xT = xT_ref[...]
    m = _gelu(lax.dot_general(w1_ref[...], xT, cT,
                              preferred_element_type=jnp.float32)
              + b1_ref[...])
    m = _gelu(lax.dot_general(w2_ref[...], m, cT,
                              preferred_element_type=jnp.float32)
              + b2_ref[...])
    o_ref[...] = (lax.dot_general(m, w3_ref[...], cT,
                                  preferred_element_type=jnp.float32)
                  + b3_ref[...])


def _edge_mlp(h_E_T, W1_w, W1_b, W2_w, W2_b, W3_w, W3_b,
              n_edges, block_offset):
    full = lambda shape: pl.BlockSpec(shape, lambda i: (0, 0))
    return pl.pallas_call(
        _edge_body,
        grid=(n_edges // EBLK,),
        in_specs=[
            pl.BlockSpec((H + NIN, EBLK), lambda i: (0, i + block_offset)),
            full((H + NIN, H)), full((H, 1)),
            full((H, H)), full((H, 1)),
            full((H, H)), full((1, H)),
        ],
        out_specs=pl.BlockSpec((EBLK, H), lambda i: (i, 0)),
        out_shape=jax.ShapeDtypeStruct((n_edges, H), jnp.float32),
    )(h_E_T, W1_w, W1_b.reshape(H, 1), W2_w, W2_b.reshape(H, 1),
      W3_w, W3_b.reshape(1, H))


@functools.lru_cache(maxsize=2)
def _make_scatter_sum(nchunks):
    mesh = plsc.VectorSubcoreMesh(core_axis_name="c", subcore_axis_name="s")
    ept = nchunks * CHUNK

    @functools.partial(
        pl.kernel,
        mesh=mesh,
        out_type=jax.ShapeDtypeStruct((NUM_CORES, N_NODES, H), jnp.float32),
        scratch_types=[
            pltpu.VMEM((nchunks, CHUNK), jnp.int32),
            pltpu.VMEM((CHUNK, H), jnp.float32),
            pltpu.VMEM((CHUNK, H), jnp.float32),
            pltpu.VMEM((CHUNK, H), jnp.float32),
            pltpu.VMEM_SHARED((N_NODES, H), jnp.float32),
            pltpu.SemaphoreType.DMA,
            pltpu.SemaphoreType.DMA,
            pltpu.SemaphoreType.DMA,
            pltpu.SemaphoreType.DMA,
            pltpu.SemaphoreType.DMA,
            pltpu.SemaphoreType.DMA,
        ],
    )
    def _scatter_sum(msg_hbm, src_hbm, init_hbm, out_hbm,
                     idx_v, rows0_v, rows1_v, rows2_v, acc_sh,
                     sem_g0, sem_g1, sem_g2, sem_s0, sem_s1, sem_s2):
        c = lax.axis_index("c")
        s = lax.axis_index("s")
        wid = c * NUM_SUBCORES + s
        last = NUM_SUBCORES - 1

        @pl.when(s < last)
        def _():
            pltpu.sync_copy(init_hbm.at[c, pl.ds(s * NODE_A, NODE_A)],
                            acc_sh.at[pl.ds(s * NODE_A, NODE_A)])

        @pl.when(s == last)
        def _():
            pltpu.sync_copy(init_hbm.at[c, pl.ds(last * NODE_A, NODE_LAST)],
                            acc_sh.at[pl.ds(last * NODE_A, NODE_LAST)])

        pltpu.sync_copy(src_hbm.at[wid], idx_v)
        plsc.subcore_barrier()

        ebase = wid * ept

        def _g(j, buf, sem):
            pltpu.async_copy(
                msg_hbm.at[pl.ds(ebase + j * CHUNK, CHUNK)], buf, sem)

        def _wg(j, buf, sem):
            pltpu.make_async_copy(
                msg_hbm.at[pl.ds(ebase + j * CHUNK, CHUNK)], buf,
                sem).wait()

        def _s(j, buf, sem):
            pltpu.async_copy(buf, acc_sh.at[idx_v.at[j]], sem, add=True)

        def _ws(buf, sem):
            pltpu.make_async_copy(buf, acc_sh.at[idx_v.at[0]], sem).wait()

        bufs = ((rows0_v, sem_g0, sem_s0),
                (rows1_v, sem_g1, sem_s1),
                (rows2_v, sem_g2, sem_s2))
        for b, (buf, gsem, _sm) in enumerate(bufs):
            _g(b, buf, gsem)

        def body(p, carry):
            j = 3 * p
            for b, (buf, gsem, ssem) in enumerate(bufs):
                _wg(j + b, buf, gsem)
                _s(j + b, buf, ssem)
            for b, (buf, gsem, ssem) in enumerate(bufs):
                _ws(buf, ssem)
                _g(j + 3 + b, buf, gsem)
            return carry

        nloop = (nchunks - 3) // 3
        tail = nchunks - 3 * nloop
        lax.fori_loop(0, nloop, body, 0)
        jt = 3 * nloop
        for b in range(3):
            buf, gsem, ssem = bufs[b]
            _wg(jt + b, buf, gsem)
            _s(jt + b, buf, ssem)
        for b in range(tail - 3):
            buf, gsem, ssem = bufs[b]
            _ws(buf, ssem)
            _g(jt + 3 + b, buf, gsem)
            _wg(jt + 3 + b, buf, gsem)
            _s(jt + 3 + b, buf, ssem)
        for b in range(3):
            _ws(bufs[b][0], bufs[b][2])
        plsc.subcore_barrier()

        @pl.when(s < last)
        def _():
            pltpu.sync_copy(acc_sh.at[pl.ds(s * NODE_A, NODE_A)],
                            out_hbm.at[c, pl.ds(s * NODE_A, NODE_A)])

        @pl.when(s == last)
        def _():
            pltpu.sync_copy(acc_sh.at[pl.ds(last * NODE_A, NODE_LAST)],
                            out_hbm.at[c, pl.ds(last * NODE_A, NODE_LAST)])

    return _scatter_sum


NBLK = 2000


def _node_body(hv_ref, p_ref, d1_ref, d1b_ref, d2_ref, d2b_ref,
               g1_ref, bb1_ref, g2_ref, bb2_ref, o_ref):
    dh = (p_ref[0] + p_ref[1]) * SCALE_INV
    h = _ln(hv_ref[...] + dh, g1_ref[...], bb1_ref[...])
    y = jnp.dot(_gelu(jnp.dot(h, d1_ref[...],
                              preferred_element_type=jnp.float32)
                      + d1b_ref[...]),
                d2_ref[...], preferred_element_type=jnp.float32) + d2b_ref[...]
    o_ref[...] = _ln(h + y, g2_ref[...], bb2_ref[...])


def _node_stage(h_V, partials, d1_w, d1_b, d2_w, d2_b,
                ln1_g, ln1_b, ln2_g, ln2_b):
    full = lambda shape: pl.BlockSpec(shape, lambda i: (0, 0))
    return pl.pallas_call(
        _node_body,
        grid=(N_NODES // NBLK,),
        in_specs=[
            pl.BlockSpec((NBLK, H), lambda i: (i, 0)),
            pl.BlockSpec((NUM_CORES, NBLK, H), lambda i: (0, i, 0)),
            full((H, 4 * H)), full((1, 4 * H)),
            full((4 * H, H)), full((1, H)),
            full((1, H)), full((1, H)),
            full((1, H)), full((1, H)),
        ],
        out_specs=pl.BlockSpec((NBLK, H), lambda i: (i, 0)),
        out_shape=jax.ShapeDtypeStruct((N_NODES, H), jnp.float32),
    )(h_V, partials, d1_w, d1_b.reshape(1, 4 * H), d2_w, d2_b.reshape(1, H),
      ln1_g.reshape(1, H), ln1_b.reshape(1, H),
      ln2_g.reshape(1, H), ln2_b.reshape(1, H))


def kernel(h_V, h_E, edge_idx, W1_w, W1_b, W2_w, W2_b, W3_w, W3_b,
           d1_w, d1_b, d2_w, d2_b, ln1_g, ln1_b, ln2_g, ln2_b):
    e_a = NW * KA * CHUNK
    h_E_T = h_E.T
    mlp_w = (W1_w, W1_b, W2_w, W2_b, W3_w, W3_b)
    msg_a = _edge_mlp(h_E_T, *mlp_w, e_a, 0)
    msg_b = _edge_mlp(h_E_T, *mlp_w, N_EDGES - e_a, e_a // EBLK)
    src = edge_idx[0].astype(jnp.int32)
    src_a = src[:e_a].reshape(NW, KA, CHUNK)
    src_b = src[e_a:].reshape(NW, KB, CHUNK)
    init0 = jnp.zeros((NUM_CORES, N_NODES, H), jnp.float32)
    part_a = _make_scatter_sum(KA)(msg_a, src_a, init0)
    part_b = _make_scatter_sum(KB)(msg_b, src_b, part_a)
    return _node_stage(h_V, part_b, d1_w, d1_b, d2_w, d2_b,
                       ln1_g, ln1_b, ln2_g, ln2_b)


# speedup vs baseline: 6.1939x; 1.0065x over previous
import functools

import jax
import jax.numpy as jnp
from jax import lax
from jax.experimental import pallas as pl
from jax.experimental.pallas import tpu as pltpu
from jax.experimental.pallas import tpu_sc as plsc

N_NODES = 10000
N_EDGES = 320000
H = 128
NIN = 16
SCALE_INV = 1.0 / 30.0

NUM_CORES = 2
NUM_SUBCORES = 16
NW = NUM_CORES * NUM_SUBCORES
EDGES_PER_TILE = N_EDGES // NW
CHUNK = 80
KA = 70
KB = 55
NODE_A = 624
NODE_LAST = N_NODES - (NUM_SUBCORES - 1) * NODE_A

_SQRT_HALF = 0.7071067811865476


def _erf(x):
    a1 = 0.254829592
    a2 = -0.284496736
    a3 = 1.421413741
    a4 = -1.453152027
    a5 = 1.061405429
    p = 0.3275911
    ax = jnp.abs(x)
    t = 1.0 / (1.0 + p * ax)
    poly = ((((a5 * t + a4) * t + a3) * t + a2) * t + a1) * t
    y = 1.0 - poly * jnp.exp(-ax * ax)
    return jnp.where(x < 0, -y, y)


def _gelu(x):
    return 0.5 * x * (1.0 + lax.erf(x * _SQRT_HALF))


def _ln(x, g, b, eps=1e-5):
    mu = jnp.mean(x, axis=-1, keepdims=True)
    var = jnp.mean((x - mu) ** 2, axis=-1, keepdims=True)
    return (x - mu) * lax.rsqrt(var + eps) * g + b


EBLK = 3200


def _edge_body(xT_ref, w1_ref, b1_ref, w2_ref, b2_ref, w3_ref, b3_ref, o_ref):
    cT = (((0,), (0,)), ((), ()))
    xT = xT_ref[...]
    m = _gelu(lax.dot_general(w1_ref[...], xT, cT,
                              preferred_element_type=jnp.float32)
              + b1_ref[...])
    m = _gelu(lax.dot_general(w2_ref[...], m, cT,
                              preferred_element_type=jnp.float32)
              + b2_ref[...])
    o_ref[...] = (lax.dot_general(m, w3_ref[...], cT,
                                  preferred_element_type=jnp.float32)
                  + b3_ref[...])


def _edge_mlp(h_E_T, W1_w, W1_b, W2_w, W2_b, W3_w, W3_b,
              n_edges, block_offset):
    full = lambda shape: pl.BlockSpec(shape, lambda i: (0, 0))
    return pl.pallas_call(
        _edge_body,
        grid=(n_edges // EBLK,),
        in_specs=[
            pl.BlockSpec((H + NIN, EBLK), lambda i: (0, i + block_offset)),
            full((H + NIN, H)), full((H, 1)),
            full((H, H)), full((H, 1)),
            full((H, H)), full((1, H)),
        ],
        out_specs=pl.BlockSpec((EBLK, H), lambda i: (i, 0)),
        out_shape=jax.ShapeDtypeStruct((n_edges, H), jnp.float32),
    )(h_E_T, W1_w, W1_b.reshape(H, 1), W2_w, W2_b.reshape(H, 1),
      W3_w, W3_b.reshape(1, H))


@functools.lru_cache(maxsize=2)
def _make_scatter_sum(nchunks):
    mesh = plsc.VectorSubcoreMesh(core_axis_name="c", subcore_axis_name="s")
    ept = nchunks * CHUNK

    @functools.partial(
        pl.kernel,
        mesh=mesh,
        out_type=jax.ShapeDtypeStruct((NUM_CORES, N_NODES, H), jnp.float32),
        scratch_types=[
            pltpu.VMEM((nchunks, CHUNK), jnp.int32),
            pltpu.VMEM((CHUNK, H), jnp.float32),
            pltpu.VMEM((CHUNK, H), jnp.float32),
            pltpu.VMEM((CHUNK, H), jnp.float32),
            pltpu.VMEM_SHARED((N_NODES, H), jnp.float32),
            pltpu.SemaphoreType.DMA,
            pltpu.SemaphoreType.DMA,
            pltpu.SemaphoreType.DMA,
            pltpu.SemaphoreType.DMA,
            pltpu.SemaphoreType.DMA,
            pltpu.SemaphoreType.DMA,
        ],
    )
    def _scatter_sum(msg_hbm, src_hbm, init_hbm, out_hbm,
                     idx_v, rows0_v, rows1_v, rows2_v, acc_sh,
                     sem_g0, sem_g1, sem_g2, sem_s0, sem_s1, sem_s2):
        c = lax.axis_index("c")
        s = lax.axis_index("s")
        wid = c * NUM_SUBCORES + s
        last = NUM_SUBCORES - 1

        @pl.when(s < last)
        def _():
            pltpu.sync_copy(init_hbm.at[c, pl.ds(s * NODE_A, NODE_A)],
                            acc_sh.at[pl.ds(s * NODE_A, NODE_A)])

        @pl.when(s == last)
        def _():
            pltpu.sync_copy(init_hbm.at[c, pl.ds(last * NODE_A, NODE_LAST)],
                            acc_sh.at[pl.ds(last * NODE_A, NODE_LAST)])

        pltpu.sync_copy(src_hbm.at[wid], idx_v)
        plsc.subcore_barrier()

        ebase = wid * ept

        def _g(j, buf, sem):
            pltpu.async_copy(
                msg_hbm.at[pl.ds(ebase + j * CHUNK, CHUNK)], buf, sem)

        def _wg(j, buf, sem):
            pltpu.make_async_copy(
                msg_hbm.at[pl.ds(ebase + j * CHUNK, CHUNK)], buf,
                sem).wait()

        def _s(j, buf, sem):
            pltpu.async_copy(buf, acc_sh.at[idx_v.at[j]], sem, add=True)

        def _ws(buf, sem):
            pltpu.make_async_copy(buf, acc_sh.at[idx_v.at[0]], sem).wait()

        bufs = ((rows0_v, sem_g0, sem_s0),
                (rows1_v, sem_g1, sem_s1),
                (rows2_v, sem_g2, sem_s2))
        for b, (buf, gsem, _sm) in enumerate(bufs):
            _g(b, buf, gsem)

        def body(p, carry):
            j = 3 * p
            for b, (buf, gsem, ssem) in enumerate(bufs):
                _wg(j + b, buf, gsem)
                _s(j + b, buf, ssem)
            for b, (buf, gsem, ssem) in enumerate(bufs):
                _ws(buf, ssem)
                _g(j + 3 + b, buf, gsem)
            return carry

        nloop = (nchunks - 3) // 3
        tail = nchunks - 3 * nloop
        lax.fori_loop(0, nloop, body, 0)
        jt = 3 * nloop
        for b in range(3):
            buf, gsem, ssem = bufs[b]
            _wg(jt + b, buf, gsem)
            _s(jt + b, buf, ssem)
        for b in range(tail - 3):
            buf, gsem, ssem = bufs[b]
            _ws(buf, ssem)
            _g(jt + 3 + b, buf, gsem)
            _wg(jt + 3 + b, buf, gsem)
            _s(jt + 3 + b, buf, ssem)
        for b in range(3):
            _ws(bufs[b][0], bufs[b][2])
        plsc.subcore_barrier()

        @pl.when(s < last)
        def _():
            pltpu.sync_copy(acc_sh.at[pl.ds(s * NODE_A, NODE_A)],
                            out_hbm.at[c, pl.ds(s * NODE_A, NODE_A)])

        @pl.when(s == last)
        def _():
            pltpu.sync_copy(acc_sh.at[pl.ds(last * NODE_A, NODE_LAST)],
                            out_hbm.at[c, pl.ds(last * NODE_A, NODE_LAST)])

    return _scatter_sum


NBLK = 2000


def _node_body(hv_ref, p_ref, d1_ref, d1b_ref, d2_ref, d2b_ref,
               g1_ref, bb1_ref, g2_ref, bb2_ref, o_ref):
    dh = (p_ref[0] + p_ref[1]) * SCALE_INV
    h = _ln(hv_ref[...] + dh, g1_ref[...], bb1_ref[...])
    y = jnp.dot(_gelu(jnp.dot(h, d1_ref[...],
                              preferred_element_type=jnp.float32)
                      + d1b_ref[...]),
                d2_ref[...], preferred_element_type=jnp.float32) + d2b_ref[...]
    o_ref[...] = _ln(h + y, g2_ref[...], bb2_ref[...])


def _node_stage(h_V, partials, d1_w, d1_b, d2_w, d2_b,
                ln1_g, ln1_b, ln2_g, ln2_b):
    full = lambda shape: pl.BlockSpec(shape, lambda i: (0, 0))
    return pl.pallas_call(
        _node_body,
        grid=(N_NODES // NBLK,),
        in_specs=[
            pl.BlockSpec((NBLK, H), lambda i: (i, 0)),
            pl.BlockSpec((NUM_CORES, NBLK, H), lambda i: (0, i, 0)),
            full((H, 4 * H)), full((1, 4 * H)),
            full((4 * H, H)), full((1, H)),
            full((1, H)), full((1, H)),
            full((1, H)), full((1, H)),
        ],
        out_specs=pl.BlockSpec((NBLK, H), lambda i: (i, 0)),
        out_shape=jax.ShapeDtypeStruct((N_NODES, H), jnp.float32),
    )(h_V, partials, d1_w, d1_b.reshape(1, 4 * H), d2_w, d2_b.reshape(1, H),
      ln1_g.reshape(1, H), ln1_b.reshape(1, H),
      ln2_g.reshape(1, H), ln2_b.reshape(1, H))


def kernel(h_V, h_E, edge_idx, W1_w, W1_b, W2_w, W2_b, W3_w, W3_b,
           d1_w, d1_b, d2_w, d2_b, ln1_g, ln1_b, ln2_g, ln2_b):
    e_a = NW * KA * CHUNK
    h_E_T = h_E.T
    mlp_w = (W1_w, W1_b, W2_w, W2_b, W3_w, W3_b)
    msg_a = _edge_mlp(h_E_T, *mlp_w, e_a, 0)
    msg_b = _edge_mlp(h_E_T, *mlp_w, N_EDGES - e_a, e_a // EBLK)
    src = edge_idx[0].astype(jnp.int32)
    src_a = src[:e_a].reshape(NW, KA, CHUNK)
    src_b = src[e_a:].reshape(NW, KB, CHUNK)
    init0 = jnp.zeros((NUM_CORES, N_NODES, H), jnp.float32)
    part_a = _make_scatter_sum(KA)(msg_a, src_a, init0)
    part_b = _make_scatter_sum(KB)(msg_b, src_b, part_a)
    return _node_stage(h_V, part_b, d1_w, d1_b, d2_w, d2_b,
                       ln1_g, ln1_b, ln2_g, ln2_b)


# speedup vs baseline: 6.7869x; 1.0958x over previous
import functools

import jax
import jax.numpy as jnp
from jax import lax
from jax.experimental import pallas as pl
from jax.experimental.pallas import tpu as pltpu
from jax.experimental.pallas import tpu_sc as plsc

N_NODES = 10000
N_EDGES = 320000
H = 128
NIN = 16
SCALE_INV = 1.0 / 30.0

NUM_CORES = 2
NUM_SUBCORES = 16
NW = NUM_CORES * NUM_SUBCORES
EDGES_PER_TILE = N_EDGES // NW
CHUNK = 80
KA = 70
KB = 55
NODE_A = 624
NODE_LAST = N_NODES - (NUM_SUBCORES - 1) * NODE_A

_SQRT_HALF = 0.7071067811865476


def _erf(x):
    a1 = 0.254829592
    a2 = -0.284496736
    a3 = 1.421413741
    a4 = -1.453152027
    a5 = 1.061405429
    p = 0.3275911
    ax = jnp.abs(x)
    t = 1.0 / (1.0 + p * ax)
    poly = ((((a5 * t + a4) * t + a3) * t + a2) * t + a1) * t
    y = 1.0 - poly * jnp.exp(-ax * ax)
    return jnp.where(x < 0, -y, y)


def _gelu(x):
    return 0.5 * x * (1.0 + lax.erf(x * _SQRT_HALF))


def _ln(x, g, b, eps=1e-5):
    mu = jnp.mean(x, axis=-1, keepdims=True)
    var = jnp.mean((x - mu) ** 2, axis=-1, keepdims=True)
    return (x - mu) * lax.rsqrt(var + eps) * g + b


EBLK = 6400


def _edge_body(xT_ref, w1_ref, b1_ref, w2_ref, b2_ref, w3_ref, b3_ref, o_ref):
    cT = (((0,), (0,)), ((), ()))
    xT = xT_ref[...]
    m = _gelu(lax.dot_general(w1_ref[...], xT, cT,
                              preferred_element_type=jnp.float32)
              + b1_ref[...])
    m = _gelu(lax.dot_general(w2_ref[...], m, cT,
                              preferred_element_type=jnp.float32)
              + b2_ref[...])
    o_ref[...] = (lax.dot_general(m, w3_ref[...], cT,
                                  preferred_element_type=jnp.float32)
                  + b3_ref[...])


def _edge_mlp(h_E_T, W1_w, W1_b, W2_w, W2_b, W3_w, W3_b,
              n_edges, block_offset):
    full = lambda shape: pl.BlockSpec(shape, lambda i: (0, 0))
    return pl.pallas_call(
        _edge_body,
        grid=(n_edges // EBLK,),
        in_specs=[
            pl.BlockSpec((H + NIN, EBLK), lambda i: (0, i + block_offset)),
            full((H + NIN, H)), full((H, 1)),
            full((H, H)), full((H, 1)),
            full((H, H)), full((1, H)),
        ],
        out_specs=pl.BlockSpec((EBLK, H), lambda i: (i, 0)),
        out_shape=jax.ShapeDtypeStruct((n_edges, H), jnp.float32),
    )(h_E_T, W1_w, W1_b.reshape(H, 1), W2_w, W2_b.reshape(H, 1),
      W3_w, W3_b.reshape(1, H))


@functools.lru_cache(maxsize=2)
def _make_scatter_sum(nchunks):
    mesh = plsc.VectorSubcoreMesh(core_axis_name="c", subcore_axis_name="s")
    ept = nchunks * CHUNK

    @functools.partial(
        pl.kernel,
        mesh=mesh,
        out_type=jax.ShapeDtypeStruct((NUM_CORES, N_NODES, H), jnp.float32),
        scratch_types=[
            pltpu.VMEM((nchunks, CHUNK), jnp.int32),
            pltpu.VMEM((CHUNK, H), jnp.float32),
            pltpu.VMEM((CHUNK, H), jnp.float32),
            pltpu.VMEM((CHUNK, H), jnp.float32),
            pltpu.VMEM_SHARED((N_NODES, H), jnp.float32),
            pltpu.SemaphoreType.DMA,
            pltpu.SemaphoreType.DMA,
            pltpu.SemaphoreType.DMA,
            pltpu.SemaphoreType.DMA,
            pltpu.SemaphoreType.DMA,
            pltpu.SemaphoreType.DMA,
        ],
    )
    def _scatter_sum(msg_hbm, src_hbm, init_hbm, out_hbm,
                     idx_v, rows0_v, rows1_v, rows2_v, acc_sh,
                     sem_g0, sem_g1, sem_g2, sem_s0, sem_s1, sem_s2):
        c = lax.axis_index("c")
        s = lax.axis_index("s")
        wid = c * NUM_SUBCORES + s
        last = NUM_SUBCORES - 1

        @pl.when(s < last)
        def _():
            pltpu.sync_copy(init_hbm.at[c, pl.ds(s * NODE_A, NODE_A)],
                            acc_sh.at[pl.ds(s * NODE_A, NODE_A)])

        @pl.when(s == last)
        def _():
            pltpu.sync_copy(init_hbm.at[c, pl.ds(last * NODE_A, NODE_LAST)],
                            acc_sh.at[pl.ds(last * NODE_A, NODE_LAST)])

        pltpu.sync_copy(src_hbm.at[wid], idx_v)
        plsc.subcore_barrier()

        ebase = wid * ept

        def _g(j, buf, sem):
            pltpu.async_copy(
                msg_hbm.at[pl.ds(ebase + j * CHUNK, CHUNK)], buf, sem)

        def _wg(j, buf, sem):
            pltpu.make_async_copy(
                msg_hbm.at[pl.ds(ebase + j * CHUNK, CHUNK)], buf,
                sem).wait()

        def _s(j, buf, sem):
            pltpu.async_copy(buf, acc_sh.at[idx_v.at[j]], sem, add=True)

        def _ws(buf, sem):
            pltpu.make_async_copy(buf, acc_sh.at[idx_v.at[0]], sem).wait()

        bufs = ((rows0_v, sem_g0, sem_s0),
                (rows1_v, sem_g1, sem_s1),
                (rows2_v, sem_g2, sem_s2))
        for b, (buf, gsem, _sm) in enumerate(bufs):
            _g(b, buf, gsem)

        def body(p, carry):
            j = 3 * p
            for b, (buf, gsem, ssem) in enumerate(bufs):
                _wg(j + b, buf, gsem)
                _s(j + b, buf, ssem)
            for b, (buf, gsem, ssem) in enumerate(bufs):
                _ws(buf, ssem)
                _g(j + 3 + b, buf, gsem)
            return carry

        nloop = (nchunks - 3) // 3
        tail = nchunks - 3 * nloop
        lax.fori_loop(0, nloop, body, 0)
        jt = 3 * nloop
        for b in range(3):
            buf, gsem, ssem = bufs[b]
            _wg(jt + b, buf, gsem)
            _s(jt + b, buf, ssem)
        for b in range(tail - 3):
            buf, gsem, ssem = bufs[b]
            _ws(buf, ssem)
            _g(jt + 3 + b, buf, gsem)
            _wg(jt + 3 + b, buf, gsem)
            _s(jt + 3 + b, buf, ssem)
        for b in range(3):
            _ws(bufs[b][0], bufs[b][2])
        plsc.subcore_barrier()

        @pl.when(s < last)
        def _():
            pltpu.sync_copy(acc_sh.at[pl.ds(s * NODE_A, NODE_A)],
                            out_hbm.at[c, pl.ds(s * NODE_A, NODE_A)])

        @pl.when(s == last)
        def _():
            pltpu.sync_copy(acc_sh.at[pl.ds(last * NODE_A, NODE_LAST)],
                            out_hbm.at[c, pl.ds(last * NODE_A, NODE_LAST)])

    return _scatter_sum


NBLK = 2000


def _node_body(hv_ref, p_ref, d1_ref, d1b_ref, d2_ref, d2b_ref,
               g1_ref, bb1_ref, g2_ref, bb2_ref, o_ref):
    dh = (p_ref[0] + p_ref[1]) * SCALE_INV
    h = _ln(hv_ref[...] + dh, g1_ref[...], bb1_ref[...])
    y = jnp.dot(_gelu(jnp.dot(h, d1_ref[...],
                              preferred_element_type=jnp.float32)
                      + d1b_ref[...]),
                d2_ref[...], preferred_element_type=jnp.float32) + d2b_ref[...]
    o_ref[...] = _ln(h + y, g2_ref[...], bb2_ref[...])


def _node_stage(h_V, partials, d1_w, d1_b, d2_w, d2_b,
                ln1_g, ln1_b, ln2_g, ln2_b):
    full = lambda shape: pl.BlockSpec(shape, lambda i: (0, 0))
    return pl.pallas_call(
        _node_body,
        grid=(N_NODES // NBLK,),
        in_specs=[
            pl.BlockSpec((NBLK, H), lambda i: (i, 0)),
            pl.BlockSpec((NUM_CORES, NBLK, H), lambda i: (0, i, 0)),
            full((H, 4 * H)), full((1, 4 * H)),
            full((4 * H, H)), full((1, H)),
            full((1, H)), full((1, H)),
            full((1, H)), full((1, H)),
        ],
        out_specs=pl.BlockSpec((NBLK, H), lambda i: (i, 0)),
        out_shape=jax.ShapeDtypeStruct((N_NODES, H), jnp.float32),
    )(h_V, partials, d1_w, d1_b.reshape(1, 4 * H), d2_w, d2_b.reshape(1, H),
      ln1_g.reshape(1, H), ln1_b.reshape(1, H),
      ln2_g.reshape(1, H), ln2_b.reshape(1, H))


def kernel(h_V, h_E, edge_idx, W1_w, W1_b, W2_w, W2_b, W3_w, W3_b,
           d1_w, d1_b, d2_w, d2_b, ln1_g, ln1_b, ln2_g, ln2_b):
    e_a = NW * KA * CHUNK
    h_E_T = h_E.T
    mlp_w = (W1_w, W1_b, W2_w, W2_b, W3_w, W3_b)
    msg_a = _edge_mlp(h_E_T, *mlp_w, e_a, 0)
    msg_b = _edge_mlp(h_E_T, *mlp_w, N_EDGES - e_a, e_a // EBLK)
    src = edge_idx[0].astype(jnp.int32)
    src_a = src[:e_a].reshape(NW, KA, CHUNK)
    src_b = src[e_a:].reshape(NW, KB, CHUNK)
    init0 = jnp.zeros((NUM_CORES, N_NODES, H), jnp.float32)
    part_a = _make_scatter_sum(KA)(msg_a, src_a, init0)
    part_b = _make_scatter_sum(KB)(msg_b, src_b, part_a)
    return _node_stage(h_V, part_b, d1_w, d1_b, d2_w, d2_b,
                       ln1_g, ln1_b, ln2_g, ln2_b)


# speedup vs baseline: 6.8716x; 1.0125x over previous
import functools

import jax
import jax.numpy as jnp
from jax import lax
from jax.experimental import pallas as pl
from jax.experimental.pallas import tpu as pltpu
from jax.experimental.pallas import tpu_sc as plsc

N_NODES = 10000
N_EDGES = 320000
H = 128
NIN = 16
SCALE_INV = 1.0 / 30.0

NUM_CORES = 2
NUM_SUBCORES = 16
NW = NUM_CORES * NUM_SUBCORES
EDGES_PER_TILE = N_EDGES // NW
CHUNK = 80
KA = 70
KB = 55
NODE_A = 624
NODE_LAST = N_NODES - (NUM_SUBCORES - 1) * NODE_A

_SQRT_HALF = 0.7071067811865476


def _erf(x):
    a1 = 0.254829592
    a2 = -0.284496736
    a3 = 1.421413741
    a4 = -1.453152027
    a5 = 1.061405429
    p = 0.3275911
    ax = jnp.abs(x)
    t = 1.0 / (1.0 + p * ax)
    poly = ((((a5 * t + a4) * t + a3) * t + a2) * t + a1) * t
    y = 1.0 - poly * jnp.exp(-ax * ax)
    return jnp.where(x < 0, -y, y)


def _gelu(x):
    return 0.5 * x * (1.0 + lax.erf(x * _SQRT_HALF))


def _ln(x, g, b, eps=1e-5):
    mu = jnp.mean(x, axis=-1, keepdims=True)
    var = jnp.mean((x - mu) ** 2, axis=-1, keepdims=True)
    return (x - mu) * lax.rsqrt(var + eps) * g + b


EBLK = 12800


def _edge_body(xT_ref, w1_ref, b1_ref, w2_ref, b2_ref, w3_ref, b3_ref, o_ref):
    cT = (((0,), (0,)), ((), ()))
    xT = xT_ref[...]
    m = _gelu(lax.dot_general(w1_ref[...], xT, cT,
                              preferred_element_type=jnp.float32)
              + b1_ref[...])
    m = _gelu(lax.dot_general(w2_ref[...], m, cT,
                              preferred_element_type=jnp.float32)
              + b2_ref[...])
    o_ref[...] = (lax.dot_general(m, w3_ref[...], cT,
                                  preferred_element_type=jnp.float32)
                  + b3_ref[...])


def _edge_mlp(h_E_T, W1_w, W1_b, W2_w, W2_b, W3_w, W3_b,
              n_edges, block_offset):
    full = lambda shape: pl.BlockSpec(shape, lambda i: (0, 0))
    return pl.pallas_call(
        _edge_body,
        grid=(n_edges // EBLK,),
        in_specs=[
            pl.BlockSpec((H + NIN, EBLK), lambda i: (0, i + block_offset)),
            full((H + NIN, H)), full((H, 1)),
            full((H, H)), full((H, 1)),
            full((H, H)), full((1, H)),
        ],
        out_specs=pl.BlockSpec((EBLK, H), lambda i: (i, 0)),
        out_shape=jax.ShapeDtypeStruct((n_edges, H), jnp.float32),
    )(h_E_T, W1_w, W1_b.reshape(H, 1), W2_w, W2_b.reshape(H, 1),
      W3_w, W3_b.reshape(1, H))


@functools.lru_cache(maxsize=2)
def _make_scatter_sum(nchunks):
    mesh = plsc.VectorSubcoreMesh(core_axis_name="c", subcore_axis_name="s")
    ept = nchunks * CHUNK

    @functools.partial(
        pl.kernel,
        mesh=mesh,
        out_type=jax.ShapeDtypeStruct((NUM_CORES, N_NODES, H), jnp.float32),
        scratch_types=[
            pltpu.VMEM((nchunks, CHUNK), jnp.int32),
            pltpu.VMEM((CHUNK, H), jnp.float32),
            pltpu.VMEM((CHUNK, H), jnp.float32),
            pltpu.VMEM((CHUNK, H), jnp.float32),
            pltpu.VMEM_SHARED((N_NODES, H), jnp.float32),
            pltpu.SemaphoreType.DMA,
            pltpu.SemaphoreType.DMA,
            pltpu.SemaphoreType.DMA,
            pltpu.SemaphoreType.DMA,
            pltpu.SemaphoreType.DMA,
            pltpu.SemaphoreType.DMA,
        ],
    )
    def _scatter_sum(msg_hbm, src_hbm, init_hbm, out_hbm,
                     idx_v, rows0_v, rows1_v, rows2_v, acc_sh,
                     sem_g0, sem_g1, sem_g2, sem_s0, sem_s1, sem_s2):
        c = lax.axis_index("c")
        s = lax.axis_index("s")
        wid = c * NUM_SUBCORES + s
        last = NUM_SUBCORES - 1

        @pl.when(s < last)
        def _():
            pltpu.sync_copy(init_hbm.at[c, pl.ds(s * NODE_A, NODE_A)],
                            acc_sh.at[pl.ds(s * NODE_A, NODE_A)])

        @pl.when(s == last)
        def _():
            pltpu.sync_copy(init_hbm.at[c, pl.ds(last * NODE_A, NODE_LAST)],
                            acc_sh.at[pl.ds(last * NODE_A, NODE_LAST)])

        pltpu.sync_copy(src_hbm.at[wid], idx_v)
        plsc.subcore_barrier()

        ebase = wid * ept

        def _g(j, buf, sem):
            pltpu.async_copy(
                msg_hbm.at[pl.ds(ebase + j * CHUNK, CHUNK)], buf, sem)

        def _wg(j, buf, sem):
            pltpu.make_async_copy(
                msg_hbm.at[pl.ds(ebase + j * CHUNK, CHUNK)], buf,
                sem).wait()

        def _s(j, buf, sem):
            pltpu.async_copy(buf, acc_sh.at[idx_v.at[j]], sem, add=True)

        def _ws(buf, sem):
            pltpu.make_async_copy(buf, acc_sh.at[idx_v.at[0]], sem).wait()

        bufs = ((rows0_v, sem_g0, sem_s0),
                (rows1_v, sem_g1, sem_s1),
                (rows2_v, sem_g2, sem_s2))
        for b, (buf, gsem, _sm) in enumerate(bufs):
            _g(b, buf, gsem)

        def body(p, carry):
            j = 3 * p
            for b, (buf, gsem, ssem) in enumerate(bufs):
                _wg(j + b, buf, gsem)
                _s(j + b, buf, ssem)
            for b, (buf, gsem, ssem) in enumerate(bufs):
                _ws(buf, ssem)
                _g(j + 3 + b, buf, gsem)
            return carry

        nloop = (nchunks - 3) // 3
        tail = nchunks - 3 * nloop
        lax.fori_loop(0, nloop, body, 0)
        jt = 3 * nloop
        for b in range(3):
            buf, gsem, ssem = bufs[b]
            _wg(jt + b, buf, gsem)
            _s(jt + b, buf, ssem)
        for b in range(tail - 3):
            buf, gsem, ssem = bufs[b]
            _ws(buf, ssem)
            _g(jt + 3 + b, buf, gsem)
            _wg(jt + 3 + b, buf, gsem)
            _s(jt + 3 + b, buf, ssem)
        for b in range(3):
            _ws(bufs[b][0], bufs[b][2])
        plsc.subcore_barrier()

        @pl.when(s < last)
        def _():
            pltpu.sync_copy(acc_sh.at[pl.ds(s * NODE_A, NODE_A)],
                            out_hbm.at[c, pl.ds(s * NODE_A, NODE_A)])

        @pl.when(s == last)
        def _():
            pltpu.sync_copy(acc_sh.at[pl.ds(last * NODE_A, NODE_LAST)],
                            out_hbm.at[c, pl.ds(last * NODE_A, NODE_LAST)])

    return _scatter_sum


NBLK = 2000


def _node_body(hv_ref, p_ref, d1_ref, d1b_ref, d2_ref, d2b_ref,
               g1_ref, bb1_ref, g2_ref, bb2_ref, o_ref):
    dh = (p_ref[0] + p_ref[1]) * SCALE_INV
    h = _ln(hv_ref[...] + dh, g1_ref[...], bb1_ref[...])
    y = jnp.dot(_gelu(jnp.dot(h, d1_ref[...],
                              preferred_element_type=jnp.float32)
                      + d1b_ref[...]),
                d2_ref[...], preferred_element_type=jnp.float32) + d2b_ref[...]
    o_ref[...] = _ln(h + y, g2_ref[...], bb2_ref[...])


def _node_stage(h_V, partials, d1_w, d1_b, d2_w, d2_b,
                ln1_g, ln1_b, ln2_g, ln2_b):
    full = lambda shape: pl.BlockSpec(shape, lambda i: (0, 0))
    return pl.pallas_call(
        _node_body,
        grid=(N_NODES // NBLK,),
        in_specs=[
            pl.BlockSpec((NBLK, H), lambda i: (i, 0)),
            pl.BlockSpec((NUM_CORES, NBLK, H), lambda i: (0, i, 0)),
            full((H, 4 * H)), full((1, 4 * H)),
            full((4 * H, H)), full((1, H)),
            full((1, H)), full((1, H)),
            full((1, H)), full((1, H)),
        ],
        out_specs=pl.BlockSpec((NBLK, H), lambda i: (i, 0)),
        out_shape=jax.ShapeDtypeStruct((N_NODES, H), jnp.float32),
    )(h_V, partials, d1_w, d1_b.reshape(1, 4 * H), d2_w, d2_b.reshape(1, H),
      ln1_g.reshape(1, H), ln1_b.reshape(1, H),
      ln2_g.reshape(1, H), ln2_b.reshape(1, H))


def kernel(h_V, h_E, edge_idx, W1_w, W1_b, W2_w, W2_b, W3_w, W3_b,
           d1_w, d1_b, d2_w, d2_b, ln1_g, ln1_b, ln2_g, ln2_b):
    e_a = NW * KA * CHUNK
    h_E_T = h_E.T
    mlp_w = (W1_w, W1_b, W2_w, W2_b, W3_w, W3_b)
    msg_a = _edge_mlp(h_E_T, *mlp_w, e_a, 0)
    msg_b = _edge_mlp(h_E_T, *mlp_w, N_EDGES - e_a, e_a // EBLK)
    src = edge_idx[0].astype(jnp.int32)
    src_a = src[:e_a].reshape(NW, KA, CHUNK)
    src_b = src[e_a:].reshape(NW, KB, CHUNK)
    init0 = jnp.zeros((NUM_CORES, N_NODES, H), jnp.float32)
    part_a = _make_scatter_sum(KA)(msg_a, src_a, init0)
    part_b = _make_scatter_sum(KB)(msg_b, src_b, part_a)
    return _node_stage(h_V, part_b, d1_w, d1_b, d2_w, d2_b,
                       ln1_g, ln1_b, ln2_g, ln2_b)
